# Initial kernel scaffold; baseline (speedup 1.0000x reference)
import jax
import jax.numpy as jnp
from jax.experimental import pallas as pl


def kernel(x, W, w_comb, edge_vals, edge_index):
    raise NotImplementedError("write your pallas kernel here")


# speedup vs baseline: 3.0730x; 3.0730x over previous
import functools

import jax
import jax.numpy as jnp
from jax import lax
from jax.experimental import pallas as pl
from jax.experimental.pallas import tpu as pltpu
from jax.experimental.pallas import tpu_sc as plsc

NC = 2
NS = 16
NW = NC * NS
L = 16
BATCH = 128
ROW_BLK = 1000


def _matmul_body(s_ref, x_ref, w_ref, o_ref):
    o_ref[...] = jnp.dot(x_ref[...], w_ref[...],
                         preferred_element_type=jnp.float32) * s_ref[0]


def _matmul(x, W, s):
    n, d_in = x.shape
    d_out = W.shape[1]
    grid = n // ROW_BLK
    return pl.pallas_call(
        _matmul_body,
        grid=(grid,),
        in_specs=[
            pl.BlockSpec(memory_space=pltpu.SMEM),
            pl.BlockSpec((ROW_BLK, d_in), lambda i: (i, 0)),
            pl.BlockSpec((d_in, d_out), lambda i: (0, 0)),
        ],
        out_specs=pl.BlockSpec((ROW_BLK, d_out), lambda i: (i, 0)),
        out_shape=jax.ShapeDtypeStruct((n, d_out), jnp.float32),
    )(s, x, W)


def _fin_body(p_ref, o_ref):
    o_ref[...] = jnp.maximum(p_ref[0] + p_ref[1], 0.0)


def _finalize(p):
    _, n, d = p.shape
    grid = n // ROW_BLK
    return pl.pallas_call(
        _fin_body,
        grid=(grid,),
        in_specs=[pl.BlockSpec((NC, ROW_BLK, d), lambda i: (0, i, 0))],
        out_specs=pl.BlockSpec((ROW_BLK, d), lambda i: (i, 0)),
        out_shape=jax.ShapeDtypeStruct((n, d), jnp.float32),
    )(p)


def _sc_scatter(pre_sup, rows2, cols2, vals2, nb):
    n, d = pre_sup.shape
    rpt = (n // NS) // 8 * 8
    last_rows = n - (NS - 1) * rpt
    n_full = rpt // BATCH
    rem = rpt - n_full * BATCH
    n_full_last = last_rows // BATCH
    rem_last = last_rows - n_full_last * BATCH
    assert rem % 8 == 0 and rem_last % 8 == 0
    mesh = plsc.VectorSubcoreMesh(core_axis_name="c", subcore_axis_name="s",
                                  num_cores=NC, num_subcores=NS)

    @functools.partial(
        pl.kernel,
        out_type=jax.ShapeDtypeStruct((NC, n, d), jnp.float32),
        mesh=mesh,
        scratch_types=[
            pltpu.VMEM((nb, BATCH), jnp.int32),
            pltpu.VMEM((nb, BATCH), jnp.int32),
            pltpu.VMEM((nb, BATCH), jnp.float32),
            pltpu.VMEM((BATCH, d), jnp.float32),
            pltpu.VMEM_SHARED((n, d), jnp.float32),
            pltpu.SemaphoreType.DMA,
        ],
    )
    def sc_kernel(pre_hbm, rows_hbm, cols_hbm, vals_hbm, out_hbm,
                  col_v, row_v, val_v, gbuf, acc, sem):
        cid = lax.axis_index("c")
        sid = lax.axis_index("s")
        wid = cid * NS + sid
        b0 = wid * nb
        pltpu.sync_copy(cols_hbm.at[pl.ds(b0, nb)], col_v)
        pltpu.sync_copy(rows_hbm.at[pl.ds(b0, nb)], row_v)
        pltpu.sync_copy(vals_hbm.at[pl.ds(b0, nb)], val_v)

        zeros = jnp.zeros((L,), jnp.float32)

        def zrow(i, _):
            for c in range(d // L):
                gbuf[i, pl.ds(c * L, L)] = zeros
            return 0

        lax.fori_loop(0, BATCH, zrow, 0)
        r0 = sid * rpt
        is_last = sid == NS - 1
        nf = jnp.where(is_last, n_full_last, n_full)

        def zchunk(j, _):
            pltpu.sync_copy(gbuf, acc.at[pl.ds(r0 + j * BATCH, BATCH)])
            return 0

        lax.fori_loop(0, nf, zchunk, 0)
        if rem:
            @pl.when(jnp.logical_not(is_last))
            def _():
                pltpu.sync_copy(gbuf.at[pl.ds(0, rem)],
                                acc.at[pl.ds(r0 + n_full * BATCH, rem)])
        if rem_last:
            @pl.when(is_last)
            def _():
                pltpu.sync_copy(
                    gbuf.at[pl.ds(0, rem_last)],
                    acc.at[pl.ds(r0 + n_full_last * BATCH, rem_last)])
        plsc.subcore_barrier()

        def batch_body(b, _):
            pltpu.async_copy(pre_hbm.at[col_v.at[b]], gbuf, sem).wait()

            def group_body(g, _):
                vv = val_v[b, pl.ds(g * L, L)]
                for j in range(L):
                    v = vv[j]
                    e = g * L + j
                    for c in range(d // L):
                        sl = pl.ds(c * L, L)
                        gbuf[e, sl] = gbuf[e, sl] * v
                return 0

            lax.fori_loop(0, BATCH // L, group_body, 0)
            pltpu.sync_copy(gbuf, acc.at[row_v.at[b]], add=True)
            return 0

        lax.fori_loop(0, nb, batch_body, 0)
        plsc.subcore_barrier()

        def wchunk(j, _):
            sl = pl.ds(r0 + j * BATCH, BATCH)
            pltpu.sync_copy(acc.at[sl], gbuf)
            pltpu.sync_copy(gbuf, out_hbm.at[cid, sl])
            return 0

        lax.fori_loop(0, nf, wchunk, 0)
        if rem:
            @pl.when(jnp.logical_not(is_last))
            def _():
                sl = pl.ds(r0 + n_full * BATCH, rem)
                pltpu.sync_copy(acc.at[sl], gbuf.at[pl.ds(0, rem)])
                pltpu.sync_copy(gbuf.at[pl.ds(0, rem)], out_hbm.at[cid, sl])
        if rem_last:
            @pl.when(is_last)
            def _():
                sl = pl.ds(r0 + n_full_last * BATCH, rem_last)
                pltpu.sync_copy(acc.at[sl], gbuf.at[pl.ds(0, rem_last)])
                pltpu.sync_copy(gbuf.at[pl.ds(0, rem_last)],
                                out_hbm.at[cid, sl])

    return sc_kernel(pre_sup, rows2, cols2, vals2)


def kernel(x, W, w_comb, edge_vals, edge_index):
    n, _ = x.shape
    e = edge_vals.shape[0]
    s = w_comb.reshape(1).astype(jnp.float32)
    pre = _matmul(x, W, s)

    epw = -(-e // (NW * BATCH * 8)) * BATCH * 8
    pad = epw * NW - e
    rows = jnp.pad(edge_index[0], (0, pad))
    cols = jnp.pad(edge_index[1], (0, pad))
    vals = jnp.pad(edge_vals, (0, pad))
    nb = epw // BATCH
    rows2 = rows.reshape(NW * nb, BATCH)
    cols2 = cols.reshape(NW * nb, BATCH)
    vals2 = vals.reshape(NW * nb, BATCH)

    partials = _sc_scatter(pre, rows2, cols2, vals2, nb)
    return _finalize(partials)


# speedup vs baseline: 3.7263x; 1.2126x over previous
import functools

import jax
import jax.numpy as jnp
from jax import lax
from jax.experimental import pallas as pl
from jax.experimental.pallas import tpu as pltpu
from jax.experimental.pallas import tpu_sc as plsc

NC = 2
NS = 16
NW = NC * NS
L = 16
BATCH = 128
ROW_BLK = 1000


def _matmul_body(s_ref, x_ref, w_ref, o_ref):
    o_ref[...] = jnp.dot(x_ref[...], w_ref[...],
                         preferred_element_type=jnp.float32) * s_ref[0]


def _matmul(x, W, s):
    n, d_in = x.shape
    d_out = W.shape[1]
    grid = n // ROW_BLK
    return pl.pallas_call(
        _matmul_body,
        grid=(grid,),
        in_specs=[
            pl.BlockSpec(memory_space=pltpu.SMEM),
            pl.BlockSpec((ROW_BLK, d_in), lambda i: (i, 0)),
            pl.BlockSpec((d_in, d_out), lambda i: (0, 0)),
        ],
        out_specs=pl.BlockSpec((ROW_BLK, d_out), lambda i: (i, 0)),
        out_shape=jax.ShapeDtypeStruct((n, d_out), jnp.float32),
    )(s, x, W)


def _fin_body(p_ref, o_ref):
    o_ref[...] = jnp.maximum(p_ref[0] + p_ref[1], 0.0)


def _finalize(p):
    _, n, d = p.shape
    grid = n // ROW_BLK
    return pl.pallas_call(
        _fin_body,
        grid=(grid,),
        in_specs=[pl.BlockSpec((NC, ROW_BLK, d), lambda i: (0, i, 0))],
        out_specs=pl.BlockSpec((ROW_BLK, d), lambda i: (i, 0)),
        out_shape=jax.ShapeDtypeStruct((n, d), jnp.float32),
    )(p)


def _sc_scatter(pre_sup, rows2, cols2, vals2, nb):
    n, d = pre_sup.shape
    rpt = (n // NS) // 8 * 8
    last_rows = n - (NS - 1) * rpt
    n_full = rpt // BATCH
    rem = rpt - n_full * BATCH
    n_full_last = last_rows // BATCH
    rem_last = last_rows - n_full_last * BATCH
    assert rem % 8 == 0 and rem_last % 8 == 0
    mesh = plsc.VectorSubcoreMesh(core_axis_name="c", subcore_axis_name="s",
                                  num_cores=NC, num_subcores=NS)

    assert nb % 8 == 0
    nb2 = nb // 2

    @functools.partial(
        pl.kernel,
        out_type=jax.ShapeDtypeStruct((NC, n, d), jnp.float32),
        mesh=mesh,
        scratch_types=[
            pltpu.VMEM((nb2, BATCH), jnp.int32),
            pltpu.VMEM((nb2, BATCH), jnp.int32),
            pltpu.VMEM((nb2, BATCH), jnp.float32),
            pltpu.VMEM((2, BATCH, d), jnp.float32),
            pltpu.VMEM_SHARED((n, d), jnp.float32),
            pltpu.SemaphoreType.DMA,
            pltpu.SemaphoreType.DMA,
        ],
    )
    def sc_kernel(pre_hbm, rows_hbm, cols_hbm, vals_hbm, out_hbm,
                  col_v, row_v, val_v, gbuf, acc, sem0, sem1):
        cid = lax.axis_index("c")
        sid = lax.axis_index("s")
        wid = cid * NS + sid

        zeros = jnp.zeros((L,), jnp.float32)

        def zrow(i, _):
            for c in range(d // L):
                gbuf[0, i, pl.ds(c * L, L)] = zeros
            return 0

        lax.fori_loop(0, BATCH, zrow, 0)
        r0 = sid * rpt
        is_last = sid == NS - 1
        nf = jnp.where(is_last, n_full_last, n_full)

        def zchunk(j, _):
            pltpu.sync_copy(gbuf.at[0], acc.at[pl.ds(r0 + j * BATCH, BATCH)])
            return 0

        lax.fori_loop(0, nf, zchunk, 0)
        if rem:
            @pl.when(jnp.logical_not(is_last))
            def _():
                pltpu.sync_copy(gbuf.at[0, pl.ds(0, rem)],
                                acc.at[pl.ds(r0 + n_full * BATCH, rem)])
        if rem_last:
            @pl.when(is_last)
            def _():
                pltpu.sync_copy(
                    gbuf.at[0, pl.ds(0, rem_last)],
                    acc.at[pl.ds(r0 + n_full_last * BATCH, rem_last)])
        plsc.subcore_barrier()

        sems = (sem0, sem1)

        def pair_body(i, _):
            i2 = i * 2
            for ph in range(2):
                b = i2 + ph
                nxt = 1 - ph

                @pl.when(b + 1 < nb2)
                def _():
                    pltpu.async_copy(pre_hbm.at[col_v.at[b + 1]],
                                     gbuf.at[nxt], sems[nxt])

                pltpu.make_async_copy(pre_hbm.at[col_v.at[b]],
                                      gbuf.at[ph], sems[ph]).wait()

                def group_body(g, _):
                    vv = val_v[b, pl.ds(g * L, L)]
                    for j in range(L):
                        v = vv[j]
                        e = g * L + j
                        for c in range(d // L):
                            sl = pl.ds(c * L, L)
                            gbuf[ph, e, sl] = gbuf[ph, e, sl] * v
                    return 0

                lax.fori_loop(0, BATCH // L, group_body, 0)
                pltpu.sync_copy(gbuf.at[ph], acc.at[row_v.at[b]], add=True)
            return 0

        for h in range(2):
            b0 = wid * nb + h * nb2
            pltpu.sync_copy(cols_hbm.at[pl.ds(b0, nb2)], col_v)
            pltpu.sync_copy(rows_hbm.at[pl.ds(b0, nb2)], row_v)
            pltpu.sync_copy(vals_hbm.at[pl.ds(b0, nb2)], val_v)
            pltpu.async_copy(pre_hbm.at[col_v.at[0]], gbuf.at[0], sem0)
            lax.fori_loop(0, nb2 // 2, pair_body, 0)
        plsc.subcore_barrier()

        def wchunk(j, _):
            sl = pl.ds(r0 + j * BATCH, BATCH)
            pltpu.sync_copy(acc.at[sl], gbuf.at[0])
            pltpu.sync_copy(gbuf.at[0], out_hbm.at[cid, sl])
            return 0

        lax.fori_loop(0, nf, wchunk, 0)
        if rem:
            @pl.when(jnp.logical_not(is_last))
            def _():
                sl = pl.ds(r0 + n_full * BATCH, rem)
                pltpu.sync_copy(acc.at[sl], gbuf.at[0, pl.ds(0, rem)])
                pltpu.sync_copy(gbuf.at[0, pl.ds(0, rem)], out_hbm.at[cid, sl])
        if rem_last:
            @pl.when(is_last)
            def _():
                sl = pl.ds(r0 + n_full_last * BATCH, rem_last)
                pltpu.sync_copy(acc.at[sl], gbuf.at[0, pl.ds(0, rem_last)])
                pltpu.sync_copy(gbuf.at[0, pl.ds(0, rem_last)],
                                out_hbm.at[cid, sl])

    return sc_kernel(pre_sup, rows2, cols2, vals2)


def kernel(x, W, w_comb, edge_vals, edge_index):
    n, _ = x.shape
    e = edge_vals.shape[0]
    s = w_comb.reshape(1).astype(jnp.float32)
    pre = _matmul(x, W, s)

    epw = -(-e // (NW * BATCH * 8)) * BATCH * 8
    pad = epw * NW - e
    rows = jnp.pad(edge_index[0], (0, pad))
    cols = jnp.pad(edge_index[1], (0, pad))
    vals = jnp.pad(edge_vals, (0, pad))
    nb = epw // BATCH
    rows2 = rows.reshape(NW * nb, BATCH)
    cols2 = cols.reshape(NW * nb, BATCH)
    vals2 = vals.reshape(NW * nb, BATCH)

    partials = _sc_scatter(pre, rows2, cols2, vals2, nb)
    return _finalize(partials)
